# Initial kernel scaffold; baseline (speedup 1.0000x reference)
import jax
import jax.numpy as jnp
from jax.experimental import pallas as pl


def kernel(node_feature, edge_index, W_src, b_src, W_dst, b_dst, attn, W_res, b_res, W_dense, b_dense):
    raise NotImplementedError("write your pallas kernel here")


# speedup vs baseline: 3.7423x; 3.7423x over previous
import functools

import jax
import jax.numpy as jnp
from jax import lax
from jax.experimental import pallas as pl
from jax.experimental.pallas import tpu as pltpu
from jax.experimental.pallas import tpu_sc as plsc

N_NODES = 10000
IN_FEAT = 256
OUT_FEAT = 256
HEADS = 4
HD = HEADS * OUT_FEAT
N_EDGES = 160000

NC = 2
NS = 16
NW = NC * NS
E_PAD = 160256
EPT = E_PAD // NW
EB = 16
NB = EPT // EB
N_PAD = 10240
CHUNK = 256
N_CHUNKS = N_PAD // CHUNK
ROWS_PER_TILE = CHUNK // NS
DEN_ROWS_PER_TILE = N_PAD // NS
NPL = HD // 128
CROWS = CHUNK + EB

_MM_BLK = 256

_GATHER_DNUMS = lax.GatherDimensionNumbers(
    offset_dims=(), collapsed_slice_dims=(0,), start_index_map=(0,))


def _lane_gather(v, idx):
    return lax.gather(v, idx[:, None], _GATHER_DNUMS, (1,),
                      mode=lax.GatherScatterMode.PROMISE_IN_BOUNDS)


def _lane_sum(v, iota16):
    for sh in (8, 4, 2, 1):
        v = v + _lane_gather(v, iota16 ^ sh)
    return v


def _mm1_body(x_ref, w_ref, b_ref, o_ref):
    acc = jnp.dot(x_ref[...], w_ref[...], preferred_element_type=jnp.float32)
    o_ref[...] = (acc + b_ref[...][None, :])[None]


def _feat_matmul(x, w_cat, b_cat):
    grid = (pl.cdiv(N_NODES, _MM_BLK), (2 * HD) // _MM_BLK)
    return pl.pallas_call(
        _mm1_body,
        grid=grid,
        in_specs=[
            pl.BlockSpec((_MM_BLK, IN_FEAT), lambda i, j: (i, 0)),
            pl.BlockSpec((IN_FEAT, _MM_BLK), lambda i, j: (0, j)),
            pl.BlockSpec((_MM_BLK,), lambda i, j: (j,)),
        ],
        out_specs=pl.BlockSpec((1, _MM_BLK, _MM_BLK),
                               lambda i, j: (j // HEADS, i, j % HEADS)),
        out_shape=jax.ShapeDtypeStruct((2, N_NODES, HD), jnp.float32),
    )(x, w_cat, b_cat)


def _tc2_body(num_ref, den_ref, x_ref, wres_ref, bres_ref, wd_ref, bd_ref,
              o_ref):
    num = jnp.concatenate(
        [num_ref[0, cp] + num_ref[1, cp] for cp in range(NPL)],
        axis=1)
    den = den_ref[0, :, 0:HEADS] + den_ref[1, :, 0:HEADS]
    recip = 1.0 / (den + 1e-9)
    res = jnp.dot(x_ref[...], wres_ref[...],
                  preferred_element_type=jnp.float32) + bres_ref[...][None, :]
    parts = [num[:, h * OUT_FEAT:(h + 1) * OUT_FEAT] * recip[:, h:h + 1]
             for h in range(HEADS)]
    rst = jnp.concatenate(parts, axis=1) + res
    out = jnp.dot(rst, wd_ref[...],
                  preferred_element_type=jnp.float32) + bd_ref[...][None, :]
    o_ref[...] = jnp.maximum(out, 0.0)


def _final_matmul(num, den, x, w_res, b_res, w_dense, b_dense):
    grid = (pl.cdiv(N_NODES, _MM_BLK),)
    return pl.pallas_call(
        _tc2_body,
        grid=grid,
        in_specs=[
            pl.BlockSpec((2, NPL, _MM_BLK, 128), lambda i: (0, 0, i, 0)),
            pl.BlockSpec((2, _MM_BLK, 16), lambda i: (0, i, 0)),
            pl.BlockSpec((_MM_BLK, IN_FEAT), lambda i: (i, 0)),
            pl.BlockSpec((IN_FEAT, HD), lambda i: (0, 0)),
            pl.BlockSpec((HD,), lambda i: (0,)),
            pl.BlockSpec((HD, OUT_FEAT), lambda i: (0, 0)),
            pl.BlockSpec((OUT_FEAT,), lambda i: (0,)),
        ],
        out_specs=pl.BlockSpec((_MM_BLK, OUT_FEAT), lambda i: (i, 0)),
        out_shape=jax.ShapeDtypeStruct((N_NODES, OUT_FEAT), jnp.float32),
    )(num, den, x, w_res, b_res, w_dense, b_dense)


def _sc_body(fs_hbm, fd_hbm, src_hbm, dst_hbm, attn_hbm, zrow_hbm, zden_hbm,
             num_hbm, den_hbm,
             src_v, dst_v, w_v, qe, fs_rows, fd_rows, stage, attn_v,
             idx_buf, dst_buf, wden, sem0, sem1,
             num_sp, den_sp):
    cid = lax.axis_index("c")
    sid = lax.axis_index("s")
    wid = sid * NC + cid
    base = wid * EPT
    zero16 = jnp.zeros((EB,), jnp.float32)
    iota16 = lax.broadcasted_iota(jnp.int32, (EB,), 0)

    pltpu.sync_copy(src_hbm.at[pl.ds(base, EPT)], src_v)
    pltpu.sync_copy(dst_hbm.at[pl.ds(base, EPT)], dst_v)
    pltpu.sync_copy(attn_hbm, attn_v)

    def _den_clear(i, c):
        pltpu.sync_copy(
            zden_hbm, den_sp.at[pl.ds(sid * DEN_ROWS_PER_TILE + i * 64, 64)])
        return c
    lax.fori_loop(0, DEN_ROWS_PER_TILE // 64, _den_clear, 0)

    def _wden_fill(i, c):
        wden[i, pl.ds(0, EB)] = zero16
        return c
    lax.fori_loop(0, EB, _wden_fill, 0)

    def _qe_fill(i, c):
        qe[pl.ds(i * EB, EB)] = jnp.zeros((EB,), jnp.int32)
        return c
    lax.fori_loop(0, (EPT + 2 * EB) // EB, _qe_fill, 0)
    plsc.subcore_barrier()

    def _p1_block(b, carry):
        off = b * EB
        srcs = src_v[pl.ds(off, EB)]
        dsts = dst_v[pl.ds(off, EB)]
        idx_buf[...] = srcs
        dst_buf[...] = jnp.minimum(dsts, N_NODES - 1)
        cp0 = pltpu.async_copy(fs_hbm.at[idx_buf], fs_rows, sem0)
        cp1 = pltpu.async_copy(fd_hbm.at[dst_buf], fd_rows, sem1)
        cp0.wait()
        cp1.wait()

        for h in range(HEADS):
            def _edge(j, lvec):
                def _acc(k, a):
                    d0 = h * OUT_FEAT + k * EB
                    s = fs_rows[j, pl.ds(d0, EB)] + fd_rows[j, pl.ds(d0, EB)]
                    act = jnp.maximum(s, 0.2 * s)
                    return a + act * attn_v[pl.ds(d0, EB)]
                acc = lax.fori_loop(0, OUT_FEAT // EB, _acc, zero16)
                return jnp.where(iota16 == j, _lane_sum(acc, iota16), lvec)
            lvec = lax.fori_loop(0, EB, _edge, zero16)
            wv = jnp.exp(lvec)
            w_v[pl.ds(h * EPT + off, EB)] = wv
            plsc.store_scatter(wden, [iota16, jnp.full((EB,), h, jnp.int32)],
                               wv)
        dst_buf[...] = dsts
        pltpu.sync_copy(wden, den_sp.at[dst_buf], add=True)
        return carry
    lax.fori_loop(0, NB, _p1_block, 0)
    plsc.subcore_barrier()

    pltpu.sync_copy(
        den_sp.at[pl.ds(sid * DEN_ROWS_PER_TILE, DEN_ROWS_PER_TILE)],
        den_hbm.at[cid, pl.ds(sid * DEN_ROWS_PER_TILE, DEN_ROWS_PER_TILE)])

    def _chunk(c, carry0):
        lo = c * CHUNK
        for cp in range(NPL):
            pltpu.sync_copy(
                zrow_hbm,
                num_sp.at[pl.ds(cp * CROWS + sid * ROWS_PER_TILE, EB)])
        plsc.subcore_barrier()

        def _scan(b, cnt):
            dsts = dst_v[pl.ds(b * EB, EB)]
            m = (dsts >= lo) & (dsts < lo + CHUNK)
            eids = b * EB + iota16
            mi = m.astype(jnp.int32)
            pos = cnt + plsc.cumsum(mi) - mi
            plsc.store_scatter(qe, [pos], eids, mask=m)
            return cnt + plsc.all_reduce_population_count(m)[0]
        cnt = lax.fori_loop(0, NB, _scan, 0)

        def _qblock(k, cc):
            off = k * EB
            ev = (off + iota16) < cnt
            eids = qe[pl.ds(off, EB)]
            srcs = plsc.load_gather(src_v, [eids])
            dstl = plsc.load_gather(dst_v, [eids]) - lo
            dstl = jnp.minimum(jnp.maximum(dstl, 0), CHUNK)
            dstl = jnp.where(ev, dstl, CHUNK)
            idx_buf[...] = srcs
            whs = [plsc.load_gather(w_v, [h * EPT + eids])
                   for h in range(HEADS)]
            pltpu.async_copy(fs_hbm.at[idx_buf], fs_rows, sem0).wait()

            def _edge(j, c2):
                jf = jnp.full((EB,), j, jnp.int32)
                wss = [_lane_gather(whs[h], jf) for h in range(HEADS)]
                for cp in range(NPL):
                    ws = wss[cp // 2]
                    def _scale(k2, c3):
                        d0 = cp * 128 + k2 * EB
                        stage[cp, j, pl.ds(k2 * EB, EB)] = (
                            fs_rows[j, pl.ds(d0, EB)] * ws)
                        return c3
                    lax.fori_loop(0, 128 // EB, _scale, 0)
                return c2
            lax.fori_loop(0, EB, _edge, 0)
            for cp in range(NPL):
                dst_buf[...] = dstl + cp * CROWS
                pltpu.sync_copy(stage.at[cp], num_sp.at[dst_buf], add=True)
            return cc
        lax.fori_loop(0, (cnt + EB - 1) // EB, _qblock, 0)
        plsc.subcore_barrier()

        for cp in range(NPL):
            pltpu.sync_copy(
                num_sp.at[pl.ds(cp * CROWS + sid * ROWS_PER_TILE,
                                ROWS_PER_TILE)],
                num_hbm.at[cid, cp,
                           pl.ds(lo + sid * ROWS_PER_TILE, ROWS_PER_TILE)])
        plsc.subcore_barrier()
        return carry0
    lax.fori_loop(0, N_CHUNKS, _chunk, 0)


_sc_kernel = functools.partial(
    pl.kernel,
    out_type=(
        jax.ShapeDtypeStruct((NC, NPL, N_PAD, 128), jnp.float32),
        jax.ShapeDtypeStruct((NC, N_PAD, 16), jnp.float32),
    ),
    mesh=plsc.VectorSubcoreMesh(core_axis_name="c", subcore_axis_name="s"),
    compiler_params=pltpu.CompilerParams(needs_layout_passes=False,
                                         use_tc_tiling_on_sc=False),
    scratch_types=[
        pltpu.VMEM((EPT,), jnp.int32),
        pltpu.VMEM((EPT,), jnp.int32),
        pltpu.VMEM((HEADS * EPT,), jnp.float32),
        pltpu.VMEM((EPT + 2 * EB,), jnp.int32),
        pltpu.VMEM((EB, HD), jnp.float32),
        pltpu.VMEM((EB, HD), jnp.float32),
        pltpu.VMEM((NPL, EB, 128), jnp.float32),
        pltpu.VMEM((HD,), jnp.float32),
        pltpu.VMEM((EB,), jnp.int32),
        pltpu.VMEM((EB,), jnp.int32),
        pltpu.VMEM((EB, 16), jnp.float32),
        pltpu.SemaphoreType.DMA,
        pltpu.SemaphoreType.DMA,
        pltpu.VMEM_SHARED((NPL * CROWS, 128), jnp.float32),
        pltpu.VMEM_SHARED((N_PAD + EB, 16), jnp.float32),
    ],
)(_sc_body)


def kernel(node_feature, edge_index, W_src, b_src, W_dst, b_dst, attn,
           W_res, b_res, W_dense, b_dense):
    x = node_feature.astype(jnp.float32)
    w_cat = jnp.concatenate([W_src, W_dst], axis=1)
    b_cat = jnp.concatenate([b_src, b_dst], axis=0)
    feats = _feat_matmul(x, w_cat, b_cat)
    fs = feats[0]
    fd = feats[1]

    ei = edge_index.astype(jnp.int32)
    pad = E_PAD - N_EDGES
    src = jnp.concatenate([ei[0], jnp.zeros((pad,), jnp.int32)])
    dst = jnp.concatenate([ei[1], jnp.full((pad,), N_PAD, jnp.int32)])

    zrow = jnp.zeros((EB, 128), jnp.float32)
    zden = jnp.zeros((64, 16), jnp.float32)
    num, den = _sc_kernel(fs, fd, src, dst, attn.reshape(HD), zrow, zden)
    return _final_matmul(num, den, x, W_res, b_res, W_dense, b_dense)
